# baseline (speedup 1.0000x reference)
import functools

import jax
import jax.numpy as jnp
from jax import lax
from jax.experimental import pallas as pl
from jax.experimental.pallas import tpu as pltpu
from jax.experimental.pallas import tpu_sc as plsc

E = 8
K = 2
H = 1024
MT = 1024
T = 512
NW = 32


def _router_body(x_ref, wr_ref, br_ref, w_ref, e_ref):
    scores = jax.lax.dot_general(
        x_ref[...], wr_ref[...], (((1,), (1,)), ((), ())),
        preferred_element_type=jnp.float32,
    ) + br_ref[...]
    m = jnp.max(scores, axis=1, keepdims=True)
    p = jnp.exp(scores - m)
    p = p / jnp.sum(p, axis=1, keepdims=True)
    iota = jax.lax.broadcasted_iota(jnp.int32, p.shape, 1)
    m0 = jnp.max(p, axis=1, keepdims=True)
    a0 = jnp.min(jnp.where(p == m0, iota, E), axis=1, keepdims=True)
    p1m = jnp.where(iota == a0, -1.0, p)
    m1 = jnp.max(p1m, axis=1, keepdims=True)
    a1 = jnp.min(jnp.where(p1m == m1, iota, E), axis=1, keepdims=True)
    ws = m0 + m1
    w_ref[...] = jnp.concatenate([m0 / ws, m1 / ws], axis=1)
    e_ref[...] = jnp.concatenate([a0, a1], axis=1)


def _router(xf, Wr, br2):
    M = xf.shape[0]
    return pl.pallas_call(
        _router_body,
        grid=(M // MT,),
        in_specs=[
            pl.BlockSpec((MT, H), lambda t: (t, 0)),
            pl.BlockSpec((E, H), lambda t: (0, 0)),
            pl.BlockSpec((1, E), lambda t: (0, 0)),
        ],
        out_specs=[
            pl.BlockSpec((MT, K), lambda t: (t, 0)),
            pl.BlockSpec((MT, K), lambda t: (t, 0)),
        ],
        out_shape=[
            jax.ShapeDtypeStruct((M, K), jnp.float32),
            jax.ShapeDtypeStruct((M, K), jnp.int32),
        ],
    )(xf, Wr, br2)


def _make_gather(M, Apad):
    HP = H // 2
    rows_w = Apad // NW
    CH = 64
    nch = rows_w // CH
    mesh = plsc.VectorSubcoreMesh(core_axis_name="c", subcore_axis_name="s")

    @functools.partial(
        pl.kernel,
        out_type=jax.ShapeDtypeStruct((Apad, HP), jnp.int32),
        mesh=mesh,
        scratch_types=[
            pltpu.VMEM((CH,), jnp.int32),
            pltpu.VMEM((CH,), jnp.int32),
            pltpu.VMEM((CH, HP), jnp.int32),
            pltpu.VMEM((CH, HP), jnp.int32),
            pltpu.SemaphoreType.DMA,
            pltpu.SemaphoreType.DMA,
            pltpu.SemaphoreType.DMA,
            pltpu.SemaphoreType.DMA,
        ],
    )
    def gather_k(x_hbm, idx_hbm, out_hbm, i0, i1, b0, b1, gs0, gs1,
                 os0, os1):
        wid = lax.axis_index("s") * 2 + lax.axis_index("c")
        base = wid * rows_w
        idxs, bufs = [i0, i1], [b0, b1]
        gsem, osem = [gs0, gs1], [os0, os1]
        gh, oh = {}, {}

        def start_gather(c):
            b = c & 1
            pltpu.sync_copy(idx_hbm.at[pl.ds(base + c * CH, CH)], idxs[b])
            gh[c] = pltpu.async_copy(x_hbm.at[idxs[b]], bufs[b], gsem[b])

        start_gather(0)
        for c in range(nch):
            b = c & 1
            if c + 1 < nch:
                if c >= 1:
                    oh[c - 1].wait()
                start_gather(c + 1)
            gh[c].wait()
            oh[c] = pltpu.async_copy(
                bufs[b], out_hbm.at[pl.ds(base + c * CH, CH)], osem[b])
        oh[nch - 2].wait()
        oh[nch - 1].wait()

    return gather_k


def _grouped_body(ue_ref, um_ref, xs_ref, we_ref, be_ref, w_ref, ys_ref):
    del um_ref
    xb = xs_ref[...]
    wb = we_ref[0].astype(jnp.bfloat16)
    y = jax.lax.dot_general(
        xb, wb, (((1,), (1,)), ((), ())),
        preferred_element_type=jnp.float32,
    ) + be_ref[0, 0]
    ys_ref[...] = y * w_ref[...]


def _grouped(xs, We, be3, w_pad2, u_e, u_m, G):
    Apad = xs.shape[0]
    return pl.pallas_call(
        _grouped_body,
        grid_spec=pltpu.PrefetchScalarGridSpec(
            num_scalar_prefetch=2,
            grid=(G,),
            in_specs=[
                pl.BlockSpec((T, H), lambda g, ue, um: (um[g], 0)),
                pl.BlockSpec((1, H, H), lambda g, ue, um: (ue[g], 0, 0)),
                pl.BlockSpec((1, 1, H), lambda g, ue, um: (ue[g], 0, 0)),
                pl.BlockSpec((T, 1), lambda g, ue, um: (um[g], 0)),
            ],
            out_specs=pl.BlockSpec((T, H), lambda g, ue, um: (um[g], 0)),
        ),
        out_shape=jax.ShapeDtypeStruct((Apad, H), jnp.float32),
    )(u_e, u_m, xs, We, be3, w_pad2)


def _make_combine(M, Apad):
    tok_w = M // NW
    CH = 16
    nch = tok_w // CH
    mesh = plsc.VectorSubcoreMesh(core_axis_name="c", subcore_axis_name="s")

    @functools.partial(
        pl.kernel,
        out_type=jax.ShapeDtypeStruct((M, H), jnp.float32),
        mesh=mesh,
        scratch_types=[
            pltpu.VMEM((CH,), jnp.int32),
            pltpu.VMEM((CH,), jnp.int32),
            pltpu.VMEM((CH,), jnp.int32),
            pltpu.VMEM((CH,), jnp.int32),
            pltpu.VMEM((CH, H), jnp.float32),
            pltpu.VMEM((CH, H), jnp.float32),
            pltpu.VMEM((CH, H), jnp.float32),
            pltpu.VMEM((CH, H), jnp.float32),
            pltpu.SemaphoreType.DMA,
            pltpu.SemaphoreType.DMA,
            pltpu.SemaphoreType.DMA,
            pltpu.SemaphoreType.DMA,
            pltpu.SemaphoreType.DMA,
            pltpu.SemaphoreType.DMA,
        ],
    )
    def combine_k(ys_hbm, pa_hbm, pb_hbm, out_hbm, ia0, ia1, ib0, ib1,
                  a0, a1, b0, b1, sa0, sa1, sb0, sb1, so0, so1):
        wid = lax.axis_index("s") * 2 + lax.axis_index("c")
        base = wid * tok_w
        ia, ib = [ia0, ia1], [ib0, ib1]
        av, bv = [a0, a1], [b0, b1]
        sa, sb, so = [sa0, sa1], [sb0, sb1], [so0, so1]
        gha, ghb, oh = {}, {}, {}

        def start_gathers(c):
            b = c & 1
            off = base + c * CH
            pltpu.sync_copy(pa_hbm.at[pl.ds(off, CH)], ia[b])
            pltpu.sync_copy(pb_hbm.at[pl.ds(off, CH)], ib[b])
            gha[c] = pltpu.async_copy(ys_hbm.at[ia[b]], av[b], sa[b])
            ghb[c] = pltpu.async_copy(ys_hbm.at[ib[b]], bv[b], sb[b])

        start_gathers(0)
        for c in range(nch):
            b = c & 1
            if c + 1 < nch:
                if c >= 1:
                    oh[c - 1].wait()
                start_gathers(c + 1)
            gha[c].wait()
            ghb[c].wait()

            def row_add(r, _):
                for j in range(H // 16):
                    s = pl.ds(j * 16, 16)
                    av[b][r, s] = av[b][r, s] + bv[b][r, s]
                return 0

            lax.fori_loop(0, CH, row_add, 0)
            oh[c] = pltpu.async_copy(
                av[b], out_hbm.at[pl.ds(base + c * CH, CH)], so[b])
        oh[nch - 2].wait()
        oh[nch - 1].wait()

    return combine_k


def kernel(x, Wr, br, We, be):
    B, S, Hx = x.shape
    M = B * S
    A = M * K
    G = A // T + E
    Apad = G * T
    xf = x.reshape(M, Hx)
    br2 = br.reshape(1, E)
    be3 = be.reshape(E, 1, Hx)

    w2, e2 = _router(xf, Wr, br2)
    ef = e2.reshape(A)
    wf = w2.reshape(A)

    oh = (ef[:, None] == jnp.arange(E)[None, :]).astype(jnp.int32)
    c = jnp.cumsum(oh, axis=0)
    counts = c[-1]
    rank = jnp.take_along_axis(c, ef[:, None], 1)[:, 0] - 1
    tpe = (counts + T - 1) // T
    poff = jnp.concatenate(
        [jnp.zeros((1,), dtype=tpe.dtype), jnp.cumsum(tpe)[:-1]]) * T
    pos = (poff[ef] + rank).astype(jnp.int32)
    used = tpe.sum()
    u_e = jnp.repeat(jnp.arange(E), tpe, total_repeat_length=G)
    u_e = jnp.where(jnp.arange(G) < used, u_e, u_e[used - 1]).astype(jnp.int32)
    u_m = jnp.minimum(jnp.arange(G), used - 1).astype(jnp.int32)
    tok_pad = jnp.zeros((Apad,), jnp.int32).at[pos].set(
        jnp.arange(A, dtype=jnp.int32) // K)
    w_pad = jnp.zeros((Apad,), jnp.float32).at[pos].set(wf)
    posA = pos[0::2]
    posB = pos[1::2]

    xp = jax.lax.bitcast_convert_type(
        xf.astype(jnp.bfloat16).reshape(M, Hx // 2, 2), jnp.int32)
    xs_p = _make_gather(M, Apad)(xp, tok_pad)
    xs = jax.lax.bitcast_convert_type(xs_p, jnp.bfloat16).reshape(Apad, Hx)
    ys = _grouped(xs, We, be3, w_pad.reshape(Apad, 1), u_e, u_m, G)
    out = _make_combine(M, Apad)(ys, posA, posB)
    return out.reshape(B, S, Hx)


# speedup vs baseline: 4.9997x; 4.9997x over previous
import jax
import jax.numpy as jnp
from jax.experimental import pallas as pl
from jax.experimental.pallas import tpu as pltpu

NUM_EXPERTS = 8
HIDDEN = 1024
MT = 1024


def _moe_body(x_ref, wr_ref, br_ref, wbig_ref, be_ref, out_ref,
              coeff_ref, xbig_ref):
    xf = x_ref[...]
    scores = jax.lax.dot_general(
        xf, wr_ref[...], (((1,), (1,)), ((), ())),
        preferred_element_type=jnp.float32,
    ) + br_ref[...]
    m = jnp.max(scores, axis=1, keepdims=True)
    p = jnp.exp(scores - m)
    p = p / jnp.sum(p, axis=1, keepdims=True)
    iota = jax.lax.broadcasted_iota(jnp.int32, p.shape, 1)
    m0 = jnp.max(p, axis=1, keepdims=True)
    a0 = jnp.min(jnp.where(p == m0, iota, NUM_EXPERTS), axis=1, keepdims=True)
    p1m = jnp.where(iota == a0, -1.0, p)
    m1 = jnp.max(p1m, axis=1, keepdims=True)
    a1 = jnp.min(jnp.where(p1m == m1, iota, NUM_EXPERTS), axis=1,
                 keepdims=True)
    wsum = m0 + m1
    coeff = (m0 * (iota == a0) + m1 * (iota == a1)) / wsum
    coeff_ref[...] = coeff

    for e in range(NUM_EXPERTS):
        xbig_ref[:, e * HIDDEN:(e + 1) * HIDDEN] = (
            xf * coeff[:, e:e + 1]).astype(jnp.bfloat16)
    y = jax.lax.dot_general(
        xbig_ref[...], wbig_ref[...], (((1,), (0,)), ((), ())),
        preferred_element_type=jnp.float32,
    )
    bias = jax.lax.dot_general(
        coeff_ref[...].astype(jnp.bfloat16), be_ref[...],
        (((1,), (0,)), ((), ())),
        preferred_element_type=jnp.float32,
    )
    out_ref[...] = y + bias


def kernel(x, Wr, br, We, be):
    B, S, H = x.shape
    M = B * S
    xf = x.reshape(M, H)
    br2 = br.reshape(1, NUM_EXPERTS)
    wbig = jnp.transpose(We, (0, 2, 1)).reshape(
        NUM_EXPERTS * H, H).astype(jnp.bfloat16)
    be_bf = be.astype(jnp.bfloat16)
    grid = (M // MT,)
    out = pl.pallas_call(
        _moe_body,
        grid=grid,
        in_specs=[
            pl.BlockSpec((MT, H), lambda t: (t, 0)),
            pl.BlockSpec((NUM_EXPERTS, H), lambda t: (0, 0)),
            pl.BlockSpec((1, NUM_EXPERTS), lambda t: (0, 0)),
            pl.BlockSpec((NUM_EXPERTS * H, H), lambda t: (0, 0)),
            pl.BlockSpec((NUM_EXPERTS, H), lambda t: (0, 0)),
        ],
        out_specs=pl.BlockSpec((MT, H), lambda t: (t, 0)),
        out_shape=jax.ShapeDtypeStruct((M, H), jnp.float32),
        scratch_shapes=[
            pltpu.VMEM((MT, NUM_EXPERTS), jnp.float32),
            pltpu.VMEM((MT, NUM_EXPERTS * HIDDEN), jnp.bfloat16),
        ],
    )(xf, Wr, br2, wbig, be_bf)
    return out.reshape(B, S, H)


# speedup vs baseline: 6.7144x; 1.3430x over previous
import jax
import jax.numpy as jnp
from jax.experimental import pallas as pl
from jax.experimental.pallas import tpu as pltpu

NUM_EXPERTS = 8
HIDDEN = 1024
MT = 512


def _moe_body(x_ref, wr_ref, br_ref, we_ref, be_ref, out_ref,
              coeff_ref, xb_ref, web_ref):
    xf = x_ref[...]
    scores = jax.lax.dot_general(
        xf, wr_ref[...], (((1,), (1,)), ((), ())),
        preferred_element_type=jnp.float32,
    ) + br_ref[...]
    m = jnp.max(scores, axis=1, keepdims=True)
    p = jnp.exp(scores - m)
    p = p / jnp.sum(p, axis=1, keepdims=True)
    iota = jax.lax.broadcasted_iota(jnp.int32, p.shape, 1)
    m0 = jnp.max(p, axis=1, keepdims=True)
    a0 = jnp.min(jnp.where(p == m0, iota, NUM_EXPERTS), axis=1, keepdims=True)
    p1m = jnp.where(iota == a0, -1.0, p)
    m1 = jnp.max(p1m, axis=1, keepdims=True)
    a1 = jnp.min(jnp.where(p1m == m1, iota, NUM_EXPERTS), axis=1,
                 keepdims=True)
    wsum = m0 + m1
    coeff_ref[...] = (m0 * (iota == a0) + m1 * (iota == a1)) / wsum

    xb_ref[...] = xf.astype(jnp.bfloat16)
    for e in range(NUM_EXPERTS):
        web_ref[...] = we_ref[e].astype(jnp.bfloat16)
        y = jax.lax.dot_general(
            xb_ref[...], web_ref[...], (((1,), (1,)), ((), ())),
            preferred_element_type=jnp.float32,
        ) + be_ref[e, 0]
        ce = coeff_ref[:, e:e + 1]
        if e == 0:
            out_ref[...] = ce * y
        else:
            out_ref[...] += ce * y


def kernel(x, Wr, br, We, be):
    B, S, H = x.shape
    M = B * S
    xf = x.reshape(M, H)
    br2 = br.reshape(1, NUM_EXPERTS)
    be3 = be.reshape(NUM_EXPERTS, 1, H)
    grid = (M // MT,)
    out = pl.pallas_call(
        _moe_body,
        grid=grid,
        in_specs=[
            pl.BlockSpec((MT, H), lambda t: (t, 0)),
            pl.BlockSpec((NUM_EXPERTS, H), lambda t: (0, 0)),
            pl.BlockSpec((1, NUM_EXPERTS), lambda t: (0, 0)),
            pl.BlockSpec((NUM_EXPERTS, H, H), lambda t: (0, 0, 0)),
            pl.BlockSpec((NUM_EXPERTS, 1, H), lambda t: (0, 0, 0)),
        ],
        out_specs=pl.BlockSpec((MT, H), lambda t: (t, 0)),
        out_shape=jax.ShapeDtypeStruct((M, H), jnp.float32),
        scratch_shapes=[
            pltpu.VMEM((MT, NUM_EXPERTS), jnp.float32),
            pltpu.VMEM((MT, HIDDEN), jnp.bfloat16),
            pltpu.VMEM((HIDDEN, HIDDEN), jnp.bfloat16),
        ],
    )(xf, Wr, br2, We, be3)
    return out.reshape(B, S, H)


# speedup vs baseline: 6.8393x; 1.0186x over previous
import jax
import jax.numpy as jnp
from jax.experimental import pallas as pl
from jax.experimental.pallas import tpu as pltpu

NUM_EXPERTS = 8
HIDDEN = 1024
MT = 1024


def _moe_body(x_ref, wr_ref, br_ref, we_ref, be_ref, out_ref,
              coeff_ref, xb_ref, web_ref):
    xf = x_ref[...]
    scores = jax.lax.dot_general(
        xf, wr_ref[...], (((1,), (1,)), ((), ())),
        preferred_element_type=jnp.float32,
    ) + br_ref[...]
    m = jnp.max(scores, axis=1, keepdims=True)
    p = jnp.exp(scores - m)
    p = p / jnp.sum(p, axis=1, keepdims=True)
    iota = jax.lax.broadcasted_iota(jnp.int32, p.shape, 1)
    m0 = jnp.max(p, axis=1, keepdims=True)
    a0 = jnp.min(jnp.where(p == m0, iota, NUM_EXPERTS), axis=1, keepdims=True)
    p1m = jnp.where(iota == a0, -1.0, p)
    m1 = jnp.max(p1m, axis=1, keepdims=True)
    a1 = jnp.min(jnp.where(p1m == m1, iota, NUM_EXPERTS), axis=1,
                 keepdims=True)
    wsum = m0 + m1
    coeff_ref[...] = (m0 * (iota == a0) + m1 * (iota == a1)) / wsum

    xb_ref[...] = xf.astype(jnp.bfloat16)
    for e in range(NUM_EXPERTS):
        web_ref[...] = we_ref[e].astype(jnp.bfloat16)
        y = jax.lax.dot_general(
            xb_ref[...], web_ref[...], (((1,), (1,)), ((), ())),
            preferred_element_type=jnp.float32,
        ) + be_ref[e, 0]
        ce = coeff_ref[:, e:e + 1]
        if e == 0:
            out_ref[...] = ce * y
        else:
            out_ref[...] += ce * y


def kernel(x, Wr, br, We, be):
    B, S, H = x.shape
    M = B * S
    xf = x.reshape(M, H)
    br2 = br.reshape(1, NUM_EXPERTS)
    be3 = be.reshape(NUM_EXPERTS, 1, H)
    grid = (M // MT,)
    out = pl.pallas_call(
        _moe_body,
        grid=grid,
        in_specs=[
            pl.BlockSpec((MT, H), lambda t: (t, 0)),
            pl.BlockSpec((NUM_EXPERTS, H), lambda t: (0, 0)),
            pl.BlockSpec((1, NUM_EXPERTS), lambda t: (0, 0)),
            pl.BlockSpec((NUM_EXPERTS, H, H), lambda t: (0, 0, 0)),
            pl.BlockSpec((NUM_EXPERTS, 1, H), lambda t: (0, 0, 0)),
        ],
        out_specs=pl.BlockSpec((MT, H), lambda t: (t, 0)),
        out_shape=jax.ShapeDtypeStruct((M, H), jnp.float32),
        scratch_shapes=[
            pltpu.VMEM((MT, NUM_EXPERTS), jnp.float32),
            pltpu.VMEM((MT, HIDDEN), jnp.bfloat16),
            pltpu.VMEM((HIDDEN, HIDDEN), jnp.bfloat16),
        ],
    )(xf, Wr, br2, We, be3)
    return out.reshape(B, S, H)
